# baseline (speedup 1.0000x reference)
import numpy as np

import jax
import jax.numpy as jnp
from jax import lax
from jax.experimental import pallas as pl
from jax.experimental.pallas import tpu as pltpu


_KH = _KW = 5


def _banded_weights(wt, C, OC, OW, KH=_KH, KW=_KW):
    WI = OW + KW - 1
    w_ = np.arange(WI)[None, :, None]
    ow_ = np.arange(OW)[None, None, :]
    j_ = np.arange(KW)[:, None, None]
    sel = jnp.asarray((w_ - ow_ == j_).astype(wt.dtype))
    wr = wt.reshape(KH, KW, C, OC)
    out = jnp.einsum("jwv,ijco->iwcvo", sel, wr)
    return out.reshape(KH, WI * C, OW * OC)


def _banded_weights_per_channel(wt, C, OC, OW, KH=_KH, KW=_KW):
    WI = OW + KW - 1
    w_ = np.arange(WI)[None, :, None]
    ow_ = np.arange(OW)[None, None, :]
    j_ = np.arange(KW)[:, None, None]
    sel = jnp.asarray((w_ - ow_ == j_).astype(wt.dtype))
    wr = wt.reshape(KH, KW, C, OC)
    out = jnp.einsum("jwv,ijco->icwvo", sel, wr)
    return out.reshape(KH * C, WI, OW * OC)


def _pool_selector(C, PW, off):
    S = np.zeros((2 * PW * C, PW * C), np.float32)
    pw = np.arange(PW)[:, None]
    c = np.arange(C)[None, :]
    S[((2 * pw + off) * C + c).ravel(), (pw * C + c).ravel()] = 1.0
    return jnp.asarray(S)


def _make_body(B):
    M = B * 32
    Mv = M - 4
    M2 = B * 16
    Mv2 = M2 - 4
    M3 = B * 8
    Mv3 = M3 - 4
    f32 = jnp.float32

    def body(x_ref, w1_ref, b1_ref, s1e_ref, s1o_ref,
             w2_ref, b2_ref, s2e_ref, s2o_ref,
             f1_ref, fb1_ref, f2_ref, fb2_ref, f3_ref, fb3_ref, o_ref):
        bf = jnp.bfloat16
        xf = x_ref[...].astype(bf).reshape(B, 96, 32)
        xcs = [xf[:, 32 * c:32 * (c + 1), :].reshape(M, 32) for c in range(3)]
        a = None
        for i in range(5):
            for c in range(3):
                p = jnp.dot(xcs[c][i:i + Mv], w1_ref[i * 3 + c],
                            preferred_element_type=f32)
                a = p if a is None else a + p
        a = jnp.maximum(a + b1_ref[...], 0.0).astype(bf)
        a = jnp.concatenate([a, jnp.zeros((4, 168), bf)], axis=0)
        cm = jnp.maximum(jnp.dot(a, s1e_ref[...], preferred_element_type=f32),
                         jnp.dot(a, s1o_ref[...], preferred_element_type=f32))
        rm = jnp.max(cm.reshape(M2, 2, 84), axis=1).astype(bf)
        a2 = jnp.dot(rm[0:Mv2], w2_ref[0], preferred_element_type=f32)
        for i in range(1, 5):
            a2 = a2 + jnp.dot(rm[i:i + Mv2], w2_ref[i], preferred_element_type=f32)
        a2 = jnp.maximum(a2 + b2_ref[...], 0.0).astype(bf)
        a2 = jnp.concatenate([a2, jnp.zeros((4, 160), bf)], axis=0)
        cm2 = jnp.maximum(jnp.dot(a2, s2e_ref[...], preferred_element_type=f32),
                          jnp.dot(a2, s2o_ref[...], preferred_element_type=f32))
        rm2 = jnp.max(cm2.reshape(M3, 2, 80), axis=1).astype(bf)
        h = jnp.dot(rm2[0:Mv3], f1_ref[0], preferred_element_type=f32)
        for p in range(1, 5):
            h = h + jnp.dot(rm2[p:p + Mv3], f1_ref[p], preferred_element_type=f32)
        h = jnp.maximum(h + fb1_ref[...], 0.0).astype(bf)
        ri = lax.broadcasted_iota(jnp.int32, (B, Mv3), 0)
        ci = lax.broadcasted_iota(jnp.int32, (B, Mv3), 1)
        sel = (ci == 8 * ri).astype(bf)
        hc = jnp.dot(sel, h, preferred_element_type=f32).astype(bf)
        h2 = jnp.maximum(jnp.dot(hc, f2_ref[...], preferred_element_type=f32)
                         + fb2_ref[...], 0.0).astype(bf)
        o_ref[...] = (jnp.dot(h2, f3_ref[...], preferred_element_type=f32)
                      + fb3_ref[...])

    return body


def kernel(c1_w, c1_b, c2_w, c2_b, fc1_w, fc1_b, fc2_w, fc2_b, fc3_w, fc3_b, x):
    N = x.shape[0]
    B = next(b for b in (128, 64, 32, 16, 8, 4, 2, 1) if N % b == 0)
    M = B * 32

    bf = jnp.bfloat16
    w1 = _banded_weights_per_channel(c1_w, 3, 6, 28).astype(bf)
    b1 = jnp.tile(c1_b.reshape(1, 6), (1, 28))
    s1e, s1o = _pool_selector(6, 14, 0).astype(bf), _pool_selector(6, 14, 1).astype(bf)
    w2 = _banded_weights(c2_w, 6, 16, 10).astype(bf)
    b2 = jnp.tile(c2_b.reshape(1, 16), (1, 10))
    s2e, s2o = _pool_selector(16, 5, 0).astype(bf), _pool_selector(16, 5, 1).astype(bf)
    f1 = fc1_w.reshape(5, 80, 120).astype(bf)
    fb1 = fc1_b.reshape(1, 120)
    fb2 = fc2_b.reshape(1, 84)
    fb3 = fc3_b.reshape(1, 10)

    x2 = x.reshape(N * 96, 32)

    res = lambda *_: (0, 0)
    resw = lambda *_: (0, 0, 0)
    out = pl.pallas_call(
        _make_body(B),
        out_shape=jax.ShapeDtypeStruct((N, 10), jnp.float32),
        grid=(N // B,),
        in_specs=[
            pl.BlockSpec((B * 96, 32), lambda b: (b, 0)),
            pl.BlockSpec((15, 32, 168), resw),
            pl.BlockSpec((1, 168), res),
            pl.BlockSpec((168, 84), res),
            pl.BlockSpec((168, 84), res),
            pl.BlockSpec((5, 84, 160), resw),
            pl.BlockSpec((1, 160), res),
            pl.BlockSpec((160, 80), res),
            pl.BlockSpec((160, 80), res),
            pl.BlockSpec((5, 80, 120), resw),
            pl.BlockSpec((1, 120), res),
            pl.BlockSpec((120, 84), res),
            pl.BlockSpec((1, 84), res),
            pl.BlockSpec((84, 10), res),
            pl.BlockSpec((1, 10), res),
        ],
        out_specs=pl.BlockSpec((B, 10), lambda b: (b, 0)),
        compiler_params=pltpu.CompilerParams(dimension_semantics=("parallel",)),
    )(x2, w1, b1, s1e, s1o, w2, b2, s2e, s2o, f1, fb1,
      fc2_w.astype(bf), fb2, fc3_w.astype(bf), fb3)
    return out


# speedup vs baseline: 1.1462x; 1.1462x over previous
import numpy as np

import jax
import jax.numpy as jnp
from jax import lax
from jax.experimental import pallas as pl
from jax.experimental.pallas import tpu as pltpu


_KH = _KW = 5


def _banded_weights(wt, C, OC, OW, KH=_KH, KW=_KW, channel_major_rows=False):
    WI = OW + KW - 1
    w_ = np.arange(WI)[None, :, None]
    ow_ = np.arange(OW)[None, None, :]
    j_ = np.arange(KW)[:, None, None]
    sel = jnp.asarray((w_ - ow_ == j_).astype(wt.dtype))
    wr = wt.reshape(KH, KW, C, OC)
    spec = "jwv,ijco->icwvo" if channel_major_rows else "jwv,ijco->iwcvo"
    return jnp.einsum(spec, sel, wr).reshape(KH, WI * C, OW * OC)


def _pool_selector(C, PW, off):
    S = np.zeros((2 * PW * C, PW * C), np.float32)
    pw = np.arange(PW)[:, None]
    c = np.arange(C)[None, :]
    S[((2 * pw + off) * C + c).ravel(), (pw * C + c).ravel()] = 1.0
    return jnp.asarray(S)


def _make_body(B):
    M = B * 32
    Mv = M - 4
    Md = M - 8
    Mf = M - 16
    f32 = jnp.float32

    def body(x0_ref, x1_ref, x2_ref, w1_ref, b1_ref, s1e_ref, s1o_ref,
             w2_ref, b2_ref, s2e_ref, s2o_ref,
             f1_ref, fb1_ref, f2_ref, fb2_ref, f3_ref, fb3_ref, o_ref):
        xcat = jnp.concatenate(
            [r[...].reshape(M, 32) for r in (x0_ref, x1_ref, x2_ref)], axis=1)
        a = jnp.dot(xcat[0:Mv], w1_ref[0], preferred_element_type=f32)
        for i in range(1, 5):
            a = a + jnp.dot(xcat[i:i + Mv], w1_ref[i], preferred_element_type=f32)
        a = jnp.maximum(a + b1_ref[...], 0.0)
        cm = jnp.maximum(jnp.dot(a, s1e_ref[...], preferred_element_type=f32),
                         jnp.dot(a, s1o_ref[...], preferred_element_type=f32))
        cm = jnp.concatenate([cm, jnp.zeros((4, 84), f32)], axis=0)
        rmax = jnp.maximum(cm, jnp.roll(cm, -1, axis=0))
        a2 = jnp.dot(rmax[0:Md], w2_ref[0], preferred_element_type=f32)
        for i in range(1, 5):
            a2 = a2 + jnp.dot(rmax[2 * i:2 * i + Md], w2_ref[i],
                              preferred_element_type=f32)
        a2 = jnp.maximum(a2 + b2_ref[...], 0.0)
        cm2 = jnp.maximum(jnp.dot(a2, s2e_ref[...], preferred_element_type=f32),
                          jnp.dot(a2, s2o_ref[...], preferred_element_type=f32))
        cm2 = jnp.concatenate([cm2, jnp.zeros((8, 80), f32)], axis=0)
        rmax2 = jnp.maximum(cm2, jnp.roll(cm2, -2, axis=0))
        h = jnp.dot(rmax2[0:Mf], f1_ref[0], preferred_element_type=f32)
        for p in range(1, 5):
            h = h + jnp.dot(rmax2[4 * p:4 * p + Mf], f1_ref[p],
                            preferred_element_type=f32)
        h = jnp.maximum(h + fb1_ref[...], 0.0)
        ri = lax.broadcasted_iota(jnp.int32, (B, Mf), 0)
        ci = lax.broadcasted_iota(jnp.int32, (B, Mf), 1)
        sel = (ci == 32 * ri).astype(f32)
        hc = jnp.dot(sel, h, preferred_element_type=f32)
        h2 = jnp.maximum(jnp.dot(hc, f2_ref[...], preferred_element_type=f32)
                         + fb2_ref[...], 0.0)
        o_ref[...] = (jnp.dot(h2, f3_ref[...], preferred_element_type=f32)
                      + fb3_ref[...])

    return body


def kernel(c1_w, c1_b, c2_w, c2_b, fc1_w, fc1_b, fc2_w, fc2_b, fc3_w, fc3_b, x):
    N = x.shape[0]
    B = next(b for b in (128, 64, 32, 16, 8, 4, 2, 1) if N % b == 0)

    w1 = _banded_weights(c1_w, 3, 6, 28, channel_major_rows=True)
    b1 = jnp.tile(c1_b.reshape(1, 6), (1, 28))
    s1e, s1o = _pool_selector(6, 14, 0), _pool_selector(6, 14, 1)
    w2 = _banded_weights(c2_w, 6, 16, 10)
    b2 = jnp.tile(c2_b.reshape(1, 16), (1, 10))
    s2e, s2o = _pool_selector(16, 5, 0), _pool_selector(16, 5, 1)
    f1 = fc1_w.reshape(5, 80, 120)
    fb1 = fc1_b.reshape(1, 120)
    fb2 = fc2_b.reshape(1, 84)
    fb3 = fc3_b.reshape(1, 10)

    res = lambda *_: (0, 0)
    resw = lambda *_: (0, 0, 0)
    out = pl.pallas_call(
        _make_body(B),
        out_shape=jax.ShapeDtypeStruct((N, 10), jnp.float32),
        grid=(N // B,),
        in_specs=[
            pl.BlockSpec((B, 1, 32, 32), lambda b: (b, 0, 0, 0)),
            pl.BlockSpec((B, 1, 32, 32), lambda b: (b, 1, 0, 0)),
            pl.BlockSpec((B, 1, 32, 32), lambda b: (b, 2, 0, 0)),
            pl.BlockSpec((5, 96, 168), resw),
            pl.BlockSpec((1, 168), res),
            pl.BlockSpec((168, 84), res),
            pl.BlockSpec((168, 84), res),
            pl.BlockSpec((5, 84, 160), resw),
            pl.BlockSpec((1, 160), res),
            pl.BlockSpec((160, 80), res),
            pl.BlockSpec((160, 80), res),
            pl.BlockSpec((5, 80, 120), resw),
            pl.BlockSpec((1, 120), res),
            pl.BlockSpec((120, 84), res),
            pl.BlockSpec((1, 84), res),
            pl.BlockSpec((84, 10), res),
            pl.BlockSpec((1, 10), res),
        ],
        out_specs=pl.BlockSpec((B, 10), lambda b: (b, 0)),
        compiler_params=pltpu.CompilerParams(dimension_semantics=("parallel",)),
    )(x, x, x, w1, b1, s1e, s1o, w2, b2, s2e, s2o,
      f1, fb1, fc2_w, fb2, fc3_w, fb3)
    return out
